# baseline (speedup 1.0000x reference)
import functools

import jax
import jax.numpy as jnp
from jax import lax
from jax.experimental import pallas as pl
from jax.experimental.pallas import tpu as pltpu
from jax.experimental.pallas import tpu_sc as plsc

B = 1024
T = 200
D = 64
LANES = 16
NW = 32
ROWS_W = (B * T) // NW
CH = 128
NCH = ROWS_W // CH
NBUF = 5


def _emb_body(idx_hbm, tok_hbm, pos_hbm, out_hbm,
              idx_v, pos_v, rows_v,
              g0, g1, g2, g3, g4, s0, s1, s2, s3, s4):
  gsem = [g0, g1, g2, g3, g4]
  ssem = [s0, s1, s2, s3, s4]
  cid = lax.axis_index("c")
  sid = lax.axis_index("s")
  wid = sid * 2 + cid
  base = wid * ROWS_W

  pltpu.sync_copy(idx_hbm.at[wid], idx_v)
  pltpu.sync_copy(pos_hbm, pos_v)

  def fire_gather(j, b):
    pltpu.async_copy(tok_hbm.at[idx_v.at[j]], rows_v.at[b], gsem[b])

  def wait_gather(b):
    pltpu.make_async_copy(tok_hbm.at[idx_v.at[0]], rows_v.at[b],
                          gsem[b]).wait()

  def fire_scatter(i, b):
    pltpu.async_copy(rows_v.at[b], out_hbm.at[pl.ds(base + i * CH, CH)],
                     ssem[b])

  def wait_scatter(b):
    pltpu.make_async_copy(rows_v.at[b], out_hbm.at[pl.ds(base, CH)],
                          ssem[b]).wait()

  for b in range(NBUF):
    fire_gather(b, b)

  def outer(i0, carry):
    for b in range(NBUF):
      i = i0 * NBUF + b
      wait_gather(b)
      t0 = lax.rem(i * CH, T)

      def add_row(r, t):
        for s in range(D // LANES):
          sl = pl.ds(s * LANES, LANES)
          rows_v[b, r, sl] = rows_v[b, r, sl] + pos_v[t, sl]
        t = t + 1
        return lax.select(t >= T, t - T, t)

      lax.fori_loop(0, CH, add_row, t0, unroll=2)
      fire_scatter(i, b)
      pb = (b - 1) % NBUF
      if b == 0:
        @pl.when(i0 >= 1)
        def _():
          wait_scatter(pb)
          fire_gather(i + NBUF - 1, pb)
      else:
        wait_scatter(pb)

        @pl.when(i + NBUF - 1 <= NCH - 1)
        def _():
          fire_gather(i + NBUF - 1, pb)
    return carry

  lax.fori_loop(0, NCH // NBUF, outer, 0)
  wait_scatter(NBUF - 1)


@functools.partial(jax.jit, donate_argnums=())
def kernel(idx, token_table, position_table):
  mesh = plsc.VectorSubcoreMesh(core_axis_name="c", subcore_axis_name="s")
  idx_r = idx.astype(jnp.int32).reshape(NW, NCH, CH)
  run = pl.kernel(
      _emb_body,
      mesh=mesh,
      compiler_params=pltpu.CompilerParams(use_tc_tiling_on_sc=False),
      out_type=jax.ShapeDtypeStruct((B * T, D), jnp.float32),
      scratch_types=[
          pltpu.VMEM((NCH, CH), jnp.int32),
          pltpu.VMEM((T, D), jnp.float32),
          pltpu.VMEM((NBUF, CH, D), jnp.float32),
      ] + [pltpu.SemaphoreType.DMA] * (2 * NBUF),
  )
  out = run(idx_r, token_table, position_table)
  return out.reshape(B, T, D)


# speedup vs baseline: 1.0315x; 1.0315x over previous
import functools

import jax
import jax.numpy as jnp
from jax import lax
from jax.experimental import pallas as pl
from jax.experimental.pallas import tpu as pltpu
from jax.experimental.pallas import tpu_sc as plsc

B = 1024
T = 200
D = 64
LANES = 16
NW = 32
ROWS_W = (B * T) // NW
CH = 128
NCH = ROWS_W // CH
NBUF = 5


def _emb_body(idx_hbm, tok_hbm, pos_hbm, out_hbm,
              idx_v, pos_v, rows_v,
              g0, g1, g2, g3, g4, s0, s1, s2, s3, s4):
  gsem = [g0, g1, g2, g3, g4]
  ssem = [s0, s1, s2, s3, s4]
  cid = lax.axis_index("c")
  sid = lax.axis_index("s")
  wid = sid * 2 + cid
  base = wid * ROWS_W

  pltpu.sync_copy(idx_hbm.at[pl.ds(base, ROWS_W)], idx_v)
  pltpu.sync_copy(pos_hbm, pos_v)

  def fire_gather(j, b):
    pltpu.async_copy(tok_hbm.at[idx_v.at[pl.ds(j * CH, CH)]],
                     rows_v.at[b], gsem[b], add=True)

  def wait_gather(b):
    pltpu.make_async_copy(tok_hbm.at[idx_v.at[pl.ds(0, CH)]],
                          rows_v.at[b], gsem[b]).wait()

  def prefill(i, b):
    t0 = lax.rem(i * CH, T)

    def fill_row(r, t):
      o = pl.multiple_of(t * D, D)
      for s in range(D // LANES):
        rows_v[b, r, pl.ds(s * LANES, LANES)] = pos_v[pl.ds(o + s * LANES,
                                                            LANES)]
      t = t + 1
      return lax.select(t >= T, t - T, t)

    lax.fori_loop(0, CH, fill_row, t0, unroll=2)

  def fire_scatter(i, b):
    pltpu.async_copy(rows_v.at[b], out_hbm.at[pl.ds(base + i * CH, CH)],
                     ssem[b])

  def wait_scatter(b):
    pltpu.make_async_copy(rows_v.at[b], out_hbm.at[pl.ds(0, CH)],
                          ssem[b]).wait()

  for b in range(NBUF):
    prefill(b, b)
    fire_gather(b, b)

  def outer(i0, carry):
    for b in range(NBUF):
      i = i0 * NBUF + b
      wait_gather(b)
      fire_scatter(i, b)
      pb = (b - 1) % NBUF
      if b == 0:
        @pl.when(i0 >= 1)
        def _():
          wait_scatter(pb)
          prefill(i + NBUF - 1, pb)
          fire_gather(i + NBUF - 1, pb)
      else:
        wait_scatter(pb)

        @pl.when(i + NBUF - 1 <= NCH - 1)
        def _():
          prefill(i + NBUF - 1, pb)
          fire_gather(i + NBUF - 1, pb)
    return carry

  lax.fori_loop(0, NCH // NBUF, outer, 0)
  wait_scatter(NBUF - 1)


@functools.partial(jax.jit, donate_argnums=())
def kernel(idx, token_table, position_table):
  mesh = plsc.VectorSubcoreMesh(core_axis_name="c", subcore_axis_name="s")
  idx_flat = idx.astype(jnp.int32).reshape(B * T)
  pos_flat = position_table.reshape(T * D)
  run = pl.kernel(
      _emb_body,
      mesh=mesh,
      compiler_params=pltpu.CompilerParams(use_tc_tiling_on_sc=False),
      out_type=jax.ShapeDtypeStruct((B * T, D), jnp.float32),
      scratch_types=[
          pltpu.VMEM((ROWS_W,), jnp.int32),
          pltpu.VMEM((T * D,), jnp.float32),
          pltpu.VMEM((NBUF, CH, D), jnp.float32),
      ] + [pltpu.SemaphoreType.DMA] * (2 * NBUF),
  )
  out = run(idx_flat, token_table, pos_flat)
  return out.reshape(B, T, D)


# speedup vs baseline: 1.1321x; 1.0975x over previous
import functools

import jax
import jax.numpy as jnp
from jax import lax
from jax.experimental import pallas as pl
from jax.experimental.pallas import tpu as pltpu
from jax.experimental.pallas import tpu_sc as plsc

B = 1024
T = 200
D = 64
LANES = 16
NW = 32
ROWS_W = (B * T) // NW
CH = 128
NCH = ROWS_W // CH
NBUF = 5


def _emb_body(idx_hbm, tok_hbm, pos_hbm, out_hbm,
              idx_v, pos_v, rows_v, zeros_v,
              g0, g1, g2, g3, g4, s0, s1, s2, s3, s4):
  gsem = [g0, g1, g2, g3, g4]
  ssem = [s0, s1, s2, s3, s4]
  cid = lax.axis_index("c")
  sid = lax.axis_index("s")
  wid = sid * 2 + cid
  base = wid * ROWS_W

  pltpu.sync_copy(idx_hbm.at[pl.ds(base, ROWS_W)], idx_v)
  pltpu.sync_copy(pos_hbm, pos_v)

  def fire_gather(j, b):
    pltpu.async_copy(tok_hbm.at[idx_v.at[pl.ds(j * CH, CH)]],
                     rows_v.at[b], gsem[b], add=True)

  def wait_gather(b):
    pltpu.make_async_copy(tok_hbm.at[idx_v.at[pl.ds(0, CH)]],
                          rows_v.at[b], gsem[b]).wait()

  def prefill(i, b):
    t0 = lax.rem(i * CH, T)

    def fill_row(r, t):
      o = pl.multiple_of(t * D, D)
      for s in range(D // LANES):
        rows_v[b, r, pl.ds(s * LANES, LANES)] = pos_v[pl.ds(o + s * LANES,
                                                            LANES)]
      t = t + 1
      return lax.select(t >= T, t - T, t)

    lax.fori_loop(0, CH, fill_row, t0, unroll=2)

  def fire_scatter(i, b):
    pltpu.async_copy(rows_v.at[b],
                     out_hbm.at[pl.ds(base + i * CH, CH), pl.ds(0, D)],
                     ssem[b])
    pltpu.async_copy(zeros_v,
                     out_hbm.at[pl.ds(base + i * CH, CH), pl.ds(D, D)],
                     ssem[b])

  def wait_scatter(b):
    pltpu.make_async_copy(rows_v.at[b],
                          out_hbm.at[pl.ds(0, CH), pl.ds(0, D)],
                          ssem[b]).wait()
    pltpu.make_async_copy(zeros_v,
                          out_hbm.at[pl.ds(0, CH), pl.ds(D, D)],
                          ssem[b]).wait()

  def zero_fill(k, carry):
    zeros_v[k // 4, pl.ds((k % 4) * LANES, LANES)] = jnp.zeros(
        (LANES,), jnp.float32)
    return carry

  lax.fori_loop(0, CH * 4, zero_fill, 0)

  for b in range(NBUF):
    prefill(b, b)
    fire_gather(b, b)

  def outer(i0, carry):
    for b in range(NBUF):
      i = i0 * NBUF + b
      wait_gather(b)
      fire_scatter(i, b)
      pb = (b - 1) % NBUF
      if b == 0:
        @pl.when(i0 >= 1)
        def _():
          wait_scatter(pb)
          prefill(i + NBUF - 1, pb)
          fire_gather(i + NBUF - 1, pb)
      else:
        wait_scatter(pb)

        @pl.when(i + NBUF - 1 <= NCH - 1)
        def _():
          prefill(i + NBUF - 1, pb)
          fire_gather(i + NBUF - 1, pb)
    return carry

  lax.fori_loop(0, NCH // NBUF, outer, 0)
  wait_scatter(NBUF - 1)


@functools.partial(jax.jit, donate_argnums=())
def kernel(idx, token_table, position_table):
  mesh = plsc.VectorSubcoreMesh(core_axis_name="c", subcore_axis_name="s")
  idx_flat = idx.astype(jnp.int32).reshape(B * T)
  pos_flat = position_table.reshape(T * D)
  run = pl.kernel(
      _emb_body,
      mesh=mesh,
      compiler_params=pltpu.CompilerParams(use_tc_tiling_on_sc=False),
      out_type=jax.ShapeDtypeStruct((B * T, 2 * D), jnp.float32),
      scratch_types=[
          pltpu.VMEM((ROWS_W,), jnp.int32),
          pltpu.VMEM((T * D,), jnp.float32),
          pltpu.VMEM((NBUF, CH, D), jnp.float32),
          pltpu.VMEM((CH, D), jnp.float32),
      ] + [pltpu.SemaphoreType.DMA] * (2 * NBUF),
  )
  out = run(idx_flat, token_table, pos_flat)
  return out[:, :D].reshape(B, T, D)


# speedup vs baseline: 1.1350x; 1.0026x over previous
import functools

import jax
import jax.numpy as jnp
from jax import lax
from jax.experimental import pallas as pl
from jax.experimental.pallas import tpu as pltpu
from jax.experimental.pallas import tpu_sc as plsc

B = 1024
T = 200
D = 64
LANES = 16
NW = 32
ROWS_W = (B * T) // NW
CH = 128
NCH = ROWS_W // CH
NBUF = 5


def _emb_body(idx_hbm, tok_hbm, pos_hbm, out_hbm,
              idx_v, pos_v, rows_v,
              g0, g1, g2, g3, g4, s0, s1, s2, s3, s4):
  gsem = [g0, g1, g2, g3, g4]
  ssem = [s0, s1, s2, s3, s4]
  cid = lax.axis_index("c")
  sid = lax.axis_index("s")
  wid = sid * 2 + cid
  base = wid * ROWS_W

  pltpu.sync_copy(idx_hbm.at[pl.ds(base, ROWS_W)], idx_v)
  pltpu.sync_copy(pos_hbm, pos_v)

  def fire_gather(j, b):
    pltpu.async_copy(tok_hbm.at[idx_v.at[pl.ds(j * CH, CH)]],
                     rows_v.at[b], gsem[b], add=True)

  def wait_gather(b):
    pltpu.make_async_copy(tok_hbm.at[idx_v.at[pl.ds(0, CH)]],
                          rows_v.at[b], gsem[b]).wait()

  def prefill(i, b):
    t0 = lax.rem(i * CH, T)

    def fill_row(r, t):
      o = pl.multiple_of(t * D, D)
      for s in range(D // LANES):
        rows_v[b, r, pl.ds(s * LANES, LANES)] = pos_v[pl.ds(o + s * LANES,
                                                            LANES)]
      t = t + 1
      return lax.select(t >= T, t - T, t)

    lax.fori_loop(0, CH, fill_row, t0, unroll=2)

  def fire_scatter(i, b):
    pltpu.async_copy(rows_v.at[b],
                     out_hbm.at[pl.ds(base + i * CH, CH), pl.ds(0, D)],
                     ssem[b])

  def wait_scatter(b):
    pltpu.make_async_copy(rows_v.at[b],
                          out_hbm.at[pl.ds(0, CH), pl.ds(0, D)],
                          ssem[b]).wait()

  for b in range(NBUF):
    prefill(b, b)
    fire_gather(b, b)

  def outer(i0, carry):
    for b in range(NBUF):
      i = i0 * NBUF + b
      wait_gather(b)
      fire_scatter(i, b)
      pb = (b - 1) % NBUF
      if b == 0:
        @pl.when(i0 >= 1)
        def _():
          wait_scatter(pb)
          prefill(i + NBUF - 1, pb)
          fire_gather(i + NBUF - 1, pb)
      else:
        wait_scatter(pb)

        @pl.when(i + NBUF - 1 <= NCH - 1)
        def _():
          prefill(i + NBUF - 1, pb)
          fire_gather(i + NBUF - 1, pb)
    return carry

  lax.fori_loop(0, NCH // NBUF, outer, 0)
  wait_scatter(NBUF - 1)


@functools.partial(jax.jit, donate_argnums=())
def kernel(idx, token_table, position_table):
  mesh = plsc.VectorSubcoreMesh(core_axis_name="c", subcore_axis_name="s")
  idx_flat = idx.astype(jnp.int32).reshape(B * T)
  pos_flat = position_table.reshape(T * D)
  run = pl.kernel(
      _emb_body,
      mesh=mesh,
      compiler_params=pltpu.CompilerParams(use_tc_tiling_on_sc=False),
      out_type=jax.ShapeDtypeStruct((B * T, 2 * D), jnp.float32),
      scratch_types=[
          pltpu.VMEM((ROWS_W,), jnp.int32),
          pltpu.VMEM((T * D,), jnp.float32),
          pltpu.VMEM((NBUF, CH, D), jnp.float32),
      ] + [pltpu.SemaphoreType.DMA] * (2 * NBUF),
  )
  out = run(idx_flat, token_table, pos_flat)
  return out[:, :D].reshape(B, T, D)


# speedup vs baseline: 1.1388x; 1.0034x over previous
import functools

import jax
import jax.numpy as jnp
from jax import lax
from jax.experimental import pallas as pl
from jax.experimental.pallas import tpu as pltpu
from jax.experimental.pallas import tpu_sc as plsc

B = 1024
T = 200
D = 64
LANES = 16
NW = 32
ROWS_W = (B * T) // NW
CH = 128
NCH = ROWS_W // CH
NBUF = 5


def _emb_body(idx_hbm, tok_hbm, pos_hbm, out_hbm,
              idx_v, pos_v, rows_v,
              g0, g1, g2, g3, g4, s0, s1, s2, s3, s4):
  gsem = [g0, g1, g2, g3, g4]
  ssem = [s0, s1, s2, s3, s4]
  cid = lax.axis_index("c")
  sid = lax.axis_index("s")
  wid = sid * 2 + cid
  base = wid * ROWS_W

  pltpu.sync_copy(idx_hbm.at[pl.ds(base, ROWS_W)], idx_v)
  pltpu.sync_copy(pos_hbm, pos_v)

  def fire_gather(j, b):
    pltpu.async_copy(tok_hbm.at[idx_v.at[pl.ds(j * CH, CH)]],
                     rows_v.at[b], gsem[b], add=True)

  def wait_gather(b):
    pltpu.make_async_copy(tok_hbm.at[idx_v.at[pl.ds(0, CH)]],
                          rows_v.at[b], gsem[b]).wait()

  def prefill(i, b):
    t0 = lax.rem(i * CH, T)

    def fill_row(r, t):
      o = pl.multiple_of(t * D, D)
      for s in range(D // LANES):
        rows_v[b, r, pl.ds(s * LANES, LANES)] = pos_v[pl.ds(o + s * LANES,
                                                            LANES)]
      t = t + 1
      return lax.select(t >= T, t - T, t)

    lax.fori_loop(0, CH, fill_row, t0, unroll=4)

  def fire_scatter(i, b):
    pltpu.async_copy(rows_v.at[b],
                     out_hbm.at[pl.ds(base + i * CH, CH), pl.ds(0, D)],
                     ssem[b])

  def wait_scatter(b):
    pltpu.make_async_copy(rows_v.at[b],
                          out_hbm.at[pl.ds(0, CH), pl.ds(0, D)],
                          ssem[b]).wait()

  for b in range(NBUF):
    prefill(b, b)
    fire_gather(b, b)

  def outer(i0, carry):
    for b in range(NBUF):
      i = i0 * NBUF + b
      wait_gather(b)
      fire_scatter(i, b)
      pb = (b - 1) % NBUF
      if b == 0:
        @pl.when(i0 >= 1)
        def _():
          wait_scatter(pb)
          prefill(i + NBUF - 1, pb)
          fire_gather(i + NBUF - 1, pb)
      else:
        wait_scatter(pb)

        @pl.when(i + NBUF - 1 <= NCH - 1)
        def _():
          prefill(i + NBUF - 1, pb)
          fire_gather(i + NBUF - 1, pb)
    return carry

  lax.fori_loop(0, NCH // NBUF, outer, 0)
  wait_scatter(NBUF - 1)


@functools.partial(jax.jit, donate_argnums=())
def kernel(idx, token_table, position_table):
  mesh = plsc.VectorSubcoreMesh(core_axis_name="c", subcore_axis_name="s")
  idx_flat = idx.astype(jnp.int32).reshape(B * T)
  pos_flat = position_table.reshape(T * D)
  run = pl.kernel(
      _emb_body,
      mesh=mesh,
      compiler_params=pltpu.CompilerParams(use_tc_tiling_on_sc=False),
      out_type=jax.ShapeDtypeStruct((B * T, 2 * D), jnp.float32),
      scratch_types=[
          pltpu.VMEM((ROWS_W,), jnp.int32),
          pltpu.VMEM((T * D,), jnp.float32),
          pltpu.VMEM((NBUF, CH, D), jnp.float32),
      ] + [pltpu.SemaphoreType.DMA] * (2 * NBUF),
  )
  out = run(idx_flat, token_table, pos_flat)
  return out[:, :D].reshape(B, T, D)


# speedup vs baseline: 1.1389x; 1.0001x over previous
import functools

import jax
import jax.numpy as jnp
from jax import lax
from jax.experimental import pallas as pl
from jax.experimental.pallas import tpu as pltpu
from jax.experimental.pallas import tpu_sc as plsc

B = 1024
T = 200
D = 64
LANES = 16
NW = 32
ROWS_W = (B * T) // NW
CH = 128
NCH = ROWS_W // CH
NBUF = 10


def _emb_body(idx_hbm, tok_hbm, pos_hbm, out_hbm,
              idx_v, pos_v, rows_v, *sems):
  gsem = list(sems[:NBUF])
  ssem = list(sems[NBUF:])
  cid = lax.axis_index("c")
  sid = lax.axis_index("s")
  wid = sid * 2 + cid
  base = wid * ROWS_W

  pltpu.sync_copy(idx_hbm.at[pl.ds(base, ROWS_W)], idx_v)
  pltpu.sync_copy(pos_hbm, pos_v)

  def fire_gather(j, b):
    pltpu.async_copy(tok_hbm.at[idx_v.at[pl.ds(j * CH, CH)]],
                     rows_v.at[b], gsem[b], add=True)

  def wait_gather(b):
    pltpu.make_async_copy(tok_hbm.at[idx_v.at[pl.ds(0, CH)]],
                          rows_v.at[b], gsem[b]).wait()

  def prefill(i, b):
    t0 = lax.rem(i * CH, T)

    def fill_row(r, t):
      o = pl.multiple_of(t * D, D)
      for s in range(D // LANES):
        rows_v[b, r, pl.ds(s * LANES, LANES)] = pos_v[pl.ds(o + s * LANES,
                                                            LANES)]
      t = t + 1
      return lax.select(t >= T, t - T, t)

    lax.fori_loop(0, CH, fill_row, t0, unroll=4)

  def fire_scatter(i, b):
    pltpu.async_copy(rows_v.at[b],
                     out_hbm.at[pl.ds(base + i * CH, CH), pl.ds(0, D)],
                     ssem[b])

  def wait_scatter(b):
    pltpu.make_async_copy(rows_v.at[b],
                          out_hbm.at[pl.ds(0, CH), pl.ds(0, D)],
                          ssem[b]).wait()

  for b in range(NBUF):
    prefill(b, b)
    fire_gather(b, b)

  def outer(i0, carry):
    for b in range(NBUF):
      i = i0 * NBUF + b
      wait_gather(b)
      fire_scatter(i, b)
      pb = (b - 1) % NBUF
      if b == 0:
        @pl.when(i0 >= 1)
        def _():
          wait_scatter(pb)
          prefill(i + NBUF - 1, pb)
          fire_gather(i + NBUF - 1, pb)
      else:
        wait_scatter(pb)

        @pl.when(i + NBUF - 1 <= NCH - 1)
        def _():
          prefill(i + NBUF - 1, pb)
          fire_gather(i + NBUF - 1, pb)
    return carry

  lax.fori_loop(0, NCH // NBUF, outer, 0)
  wait_scatter(NBUF - 1)


@functools.partial(jax.jit, donate_argnums=())
def kernel(idx, token_table, position_table):
  mesh = plsc.VectorSubcoreMesh(core_axis_name="c", subcore_axis_name="s")
  idx_flat = idx.astype(jnp.int32).reshape(B * T)
  pos_flat = position_table.reshape(T * D)
  run = pl.kernel(
      _emb_body,
      mesh=mesh,
      compiler_params=pltpu.CompilerParams(use_tc_tiling_on_sc=False),
      out_type=jax.ShapeDtypeStruct((B * T, 2 * D), jnp.float32),
      scratch_types=[
          pltpu.VMEM((ROWS_W,), jnp.int32),
          pltpu.VMEM((T * D,), jnp.float32),
          pltpu.VMEM((NBUF, CH, D), jnp.float32),
      ] + [pltpu.SemaphoreType.DMA] * (2 * NBUF),
  )
  out = run(idx_flat, token_table, pos_flat)
  return out[:, :D].reshape(B, T, D)
